# baseline (speedup 1.0000x reference)
import jax
import jax.numpy as jnp
from jax import lax
from jax.experimental import pallas as pl
from jax.experimental.pallas import tpu as pltpu
from jax.experimental.pallas import tpu_sc as plsc

T = 3
N = 10000
E = 320000
D = 128

NB = 10
BN = N // NB
NTILES = 32
EPT = E // NTILES
CH = 80
NCH = EPT // CH
CPG = 25
NG = NCH // CPG
EPG = CPG * CH
RPT = 632
NPAD = RPT * 16
RLAST = N - RPT * 15


def _tc_merge_kernel(p_ref, qp_ref, cp_ref, wi, ui, bi, wf, uf, bf,
                     wo, uo, bo, wg, ug, bg, o_ref, qn_ref, cn_ref, q_s, c_s):
    nb = pl.program_id(0)

    @pl.when(nb == 0)
    def _lstm_step():
        q = qp_ref[...]

        def gate(a_ref, b_ref, bias_ref):
            return (jnp.dot(a_ref[...], q, preferred_element_type=jnp.float32)
                    + jnp.dot(b_ref[...], q, preferred_element_type=jnp.float32)
                    + bias_ref[...])

        i = jax.nn.sigmoid(gate(wi, ui, bi))
        f = jax.nn.sigmoid(gate(wf, uf, bf))
        o = jax.nn.sigmoid(gate(wo, uo, bo))
        g = jnp.tanh(gate(wg, ug, bg))
        c = f * cp_ref[...] + i * g
        c_s[...] = c
        q_s[...] = o * jnp.tanh(c)
        qn_ref[...] = q_s[...]
        cn_ref[...] = c

    s = jnp.dot(p_ref[0] + p_ref[1], q_s[...], preferred_element_type=jnp.float32)
    o_ref[...] = jnp.where(s >= 0, s, 0.01 * s)


def _sc_scatter_kernel(src_ref, dst_ref, ewf_ref, ne_ref, zro_ref, out_ref,
                       srcg_v, dstg_v, wg_v, r0, r1, r2, zb_v, agg_sh,
                       gs0, gs1, gs2, ss0, ss1, ss2, msem):
    c = lax.axis_index("c")
    s = lax.axis_index("s")
    w_id = c * 16 + s
    rbase = s * RPT
    rows = (r0, r1, r2)
    gsems = (gs0, gs1, gs2)
    ssems = (ss0, ss1, ss2)

    pltpu.sync_copy(zro_ref, zb_v)
    for b in range(7):
        pltpu.sync_copy(zb_v, agg_sh.at[pl.ds(rbase + CH * b, CH)])
    pltpu.sync_copy(zb_v.at[pl.ds(0, RPT - 7 * CH)],
                    agg_sh.at[pl.ds(rbase + 7 * CH, RPT - 7 * CH)])
    plsc.subcore_barrier()

    def _gather(i, b):
        return pltpu.make_async_copy(
            ne_ref.at[srcg_v.at[pl.ds(i * CH, CH)]], rows[b], gsems[b])

    def _scatter(i, b):
        return pltpu.make_async_copy(rows[b], agg_sh.at[dstg_v.at[i]], ssems[b])

    def _scale(i, b):
        rb = rows[b]

        def _k16(kk, carry):
            wv = wg_v[pl.ds(i * CH + kk * 16, 16)]
            for e0 in range(16):
                we = wv[e0]
                r = kk * 16 + e0
                for j in range(8):
                    sl = pl.ds(16 * j, 16)
                    rb[r, sl] = rb[r, sl] * we
            return carry

        lax.fori_loop(0, CH // 16, _k16, 0)

    def _g_body(g, gcarry):
        off = w_id * EPT + g * EPG
        ma = pltpu.async_copy(src_ref.at[pl.ds(off, EPG)], srcg_v, msem)
        mb = pltpu.async_copy(dst_ref.at[w_id * NG + g], dstg_v, msem)
        mc = pltpu.async_copy(ewf_ref.at[pl.ds(off, EPG)], wg_v, msem)
        ma.wait()
        mb.wait()
        mc.wait()

        _gather(0, 0).start()

        def _k_body(k, kcarry):
            for cc in range(3):
                i = 3 * k + cc

                @pl.when(i >= 2)
                def _free():
                    _scatter(i - 2, (cc + 1) % 3).wait()

                @pl.when(i + 1 < CPG)
                def _pref():
                    _gather(i + 1, (cc + 1) % 3).start()

                _gather(i, cc).wait()
                _scale(i, cc)
                _scatter(i, cc).start(add=True)
            return kcarry

        lax.fori_loop(0, CPG // 3, _k_body, 0)
        _scatter(CPG - 3, 1).wait()
        _gather(CPG - 1, 0).wait()
        _scale(CPG - 1, 0)
        _scatter(CPG - 1, 0).start(add=True)
        _scatter(CPG - 2, 2).wait()
        _scatter(CPG - 1, 0).wait()
        return gcarry

    lax.fori_loop(0, NG, _g_body, 0)
    plsc.subcore_barrier()

    @pl.when(s < 15)
    def _wr_full():
        pltpu.sync_copy(agg_sh.at[pl.ds(rbase, RPT)],
                        out_ref.at[c, pl.ds(rbase, RPT)])

    @pl.when(s == 15)
    def _wr_last():
        pltpu.sync_copy(agg_sh.at[pl.ds(rbase, RLAST)],
                        out_ref.at[c, pl.ds(rbase, RLAST)])


_sc_scatter = pl.kernel(
    _sc_scatter_kernel,
    out_type=jax.ShapeDtypeStruct((2, N, D), jnp.float32),
    mesh=plsc.VectorSubcoreMesh(core_axis_name="c", subcore_axis_name="s"),
    scratch_types=[
        pltpu.VMEM((EPG,), jnp.int32),
        pltpu.VMEM((CPG, CH), jnp.int32),
        pltpu.VMEM((EPG,), jnp.float32),
        pltpu.VMEM((CH, D), jnp.float32),
        pltpu.VMEM((CH, D), jnp.float32),
        pltpu.VMEM((CH, D), jnp.float32),
        pltpu.VMEM((CH, D), jnp.float32),
        pltpu.VMEM_SHARED((NPAD, D), jnp.float32),
        pltpu.SemaphoreType.DMA,
        pltpu.SemaphoreType.DMA,
        pltpu.SemaphoreType.DMA,
        pltpu.SemaphoreType.DMA,
        pltpu.SemaphoreType.DMA,
        pltpu.SemaphoreType.DMA,
        pltpu.SemaphoreType.DMA,
    ],
)


def kernel(node_embs, edge_index, edge_weight, gcn_init_weights,
           Wi, Ui, bi, Wf, Uf, bf, Wo, Uo, bo, Wg, Ug, bg):
    zrows = jnp.zeros((CH, D), jnp.float32)
    full = pl.BlockSpec((D, D), lambda nb: (0, 0))
    merge = pl.pallas_call(
        _tc_merge_kernel,
        grid=(NB,),
        in_specs=[pl.BlockSpec((2, BN, D), lambda nb: (0, nb, 0))] + [full] * 14,
        out_specs=(pl.BlockSpec((BN, D), lambda nb: (nb, 0)), full, full),
        out_shape=(jax.ShapeDtypeStruct((N, D), jnp.float32),
                   jax.ShapeDtypeStruct((D, D), jnp.float32),
                   jax.ShapeDtypeStruct((D, D), jnp.float32)),
        scratch_shapes=[pltpu.VMEM((D, D), jnp.float32),
                        pltpu.VMEM((D, D), jnp.float32)],
    )

    parts = [
        _sc_scatter(edge_index[t, 0],
                    edge_index[t, 1].reshape(NTILES * NG, CPG, CH),
                    edge_weight[t], node_embs[t], zrows)
        for t in range(T)
    ]
    q = gcn_init_weights
    cc = jnp.zeros((D, D), jnp.float32)
    outs = []
    for t in range(T):
        o, q, cc = merge(parts[t], q, cc, Wi, Ui, bi, Wf, Uf, bf,
                         Wo, Uo, bo, Wg, Ug, bg)
        outs.append(o)
    return jnp.stack(outs, axis=0)


# speedup vs baseline: 1.0105x; 1.0105x over previous
import functools

import jax
import jax.numpy as jnp
from jax import lax
from jax.experimental import pallas as pl
from jax.experimental.pallas import tpu as pltpu
from jax.experimental.pallas import tpu_sc as plsc

T = 3
N = 10000
E = 320000
D = 128

NB = 10
BN = N // NB
NTILES = 32
EPT = E // NTILES
CH = 80
NCH = EPT // CH
CPG = 25
NG = NCH // CPG
EPG = CPG * CH
RPT = 632
NPAD = RPT * 16
RLAST = N - RPT * 15
ROWS_T = T * E // CH


def _tc_merge_kernel(p_ref, g_ref, wi, ui, bi, wf, uf, bf, wo, uo, bo,
                     wg, ug, bg, o_ref, q_s, c_s):
    t = pl.program_id(0)
    nb = pl.program_id(1)

    @pl.when(nb == 0)
    def _lstm_step():
        @pl.when(t == 0)
        def _init():
            q_s[...] = g_ref[...]
            c_s[...] = jnp.zeros_like(c_s)

        q = q_s[...]

        def gate(a_ref, b_ref, bias_ref):
            return (jnp.dot(a_ref[...], q, preferred_element_type=jnp.float32)
                    + jnp.dot(b_ref[...], q, preferred_element_type=jnp.float32)
                    + bias_ref[...])

        i = jax.nn.sigmoid(gate(wi, ui, bi))
        f = jax.nn.sigmoid(gate(wf, uf, bf))
        o = jax.nn.sigmoid(gate(wo, uo, bo))
        g = jnp.tanh(gate(wg, ug, bg))
        c = f * c_s[...] + i * g
        c_s[...] = c
        q_s[...] = o * jnp.tanh(c)

    s = jnp.dot(p_ref[0, 0] + p_ref[1, 0], q_s[...],
                preferred_element_type=jnp.float32)
    o_ref[0] = jnp.where(s >= 0, s, 0.01 * s)


def _sc_scatter_kernel(src_ref, dst_ref, ewf_ref, h_ref, zro_ref, out_ref,
                       srcg_v, dstg_v, wg_v, r0, r1, r2, zb_v, agg_sh,
                       gs0, gs1, gs2, ss0, ss1, ss2, msem):
    c = lax.axis_index("c")
    s = lax.axis_index("s")
    w_id = c * 16 + s
    rbase = s * RPT
    rows = (r0, r1, r2)
    gsems = (gs0, gs1, gs2)
    ssems = (ss0, ss1, ss2)

    pltpu.sync_copy(zro_ref, zb_v)

    def _gather(i, b):
        return pltpu.make_async_copy(
            h_ref.at[srcg_v.at[pl.ds(i * CH, CH)]], rows[b], gsems[b])

    def _scatter(i, b):
        return pltpu.make_async_copy(rows[b], agg_sh.at[dstg_v.at[i]], ssems[b])

    def _scale(i, b):
        rb = rows[b]

        def _k16(kk, carry):
            wv = wg_v[pl.ds(i * CH + kk * 16, 16)]
            for e0 in range(16):
                we = wv[e0]
                r = kk * 16 + e0
                for j in range(8):
                    sl = pl.ds(16 * j, 16)
                    rb[r, sl] = rb[r, sl] * we
            return carry

        lax.fori_loop(0, CH // 16, _k16, 0)

    def _t_body(t, tcarry):
        zcps = [pltpu.async_copy(zb_v, agg_sh.at[pl.ds(rbase + CH * b, CH)],
                                 msem)
                for b in range(7)]
        zcps.append(pltpu.async_copy(
            zb_v.at[pl.ds(0, RPT - 7 * CH)],
            agg_sh.at[pl.ds(rbase + 7 * CH, RPT - 7 * CH)], msem))
        for cp in zcps:
            cp.wait()
        plsc.subcore_barrier()

        def _g_body(g, gcarry):
            off = t * E + w_id * EPT + g * EPG
            ma = pltpu.async_copy(src_ref.at[pl.ds(off, EPG)], srcg_v, msem)
            mb = pltpu.async_copy(dst_ref.at[t * (NTILES * NG) + w_id * NG + g],
                                  dstg_v, msem)
            mc = pltpu.async_copy(ewf_ref.at[pl.ds(off, EPG)], wg_v, msem)
            ma.wait()
            mb.wait()
            mc.wait()

            _gather(0, 0).start()

            def _k_body(k, kcarry):
                for cc in range(3):
                    i = 3 * k + cc

                    @pl.when(i >= 2)
                    def _free():
                        _scatter(i - 2, (cc + 1) % 3).wait()

                    @pl.when(i + 1 < CPG)
                    def _pref():
                        _gather(i + 1, (cc + 1) % 3).start()

                    _gather(i, cc).wait()
                    _scale(i, cc)
                    _scatter(i, cc).start(add=True)
                return kcarry

            lax.fori_loop(0, CPG // 3, _k_body, 0)
            _scatter(CPG - 3, 1).wait()
            _gather(CPG - 1, 0).wait()
            _scale(CPG - 1, 0)
            _scatter(CPG - 1, 0).start(add=True)
            _scatter(CPG - 2, 2).wait()
            _scatter(CPG - 1, 0).wait()
            return gcarry

        lax.fori_loop(0, NG, _g_body, 0)
        plsc.subcore_barrier()

        @pl.when(s < 15)
        def _wr_full():
            pltpu.sync_copy(agg_sh.at[pl.ds(rbase, RPT)],
                            out_ref.at[c, t, pl.ds(rbase, RPT)])

        @pl.when(s == 15)
        def _wr_last():
            pltpu.sync_copy(agg_sh.at[pl.ds(rbase, RLAST)],
                            out_ref.at[c, t, pl.ds(rbase, RLAST)])

        return tcarry

    lax.fori_loop(0, T, _t_body, 0)


def kernel(node_embs, edge_index, edge_weight, gcn_init_weights,
           Wi, Ui, bi, Wf, Uf, bf, Wo, Uo, bo, Wg, Ug, bg):
    sc_scatter = functools.partial(
        pl.kernel,
        out_type=jax.ShapeDtypeStruct((2, T, N, D), jnp.float32),
        mesh=plsc.VectorSubcoreMesh(core_axis_name="c", subcore_axis_name="s"),
        scratch_types=[
            pltpu.VMEM((EPG,), jnp.int32),
            pltpu.VMEM((CPG, CH), jnp.int32),
            pltpu.VMEM((EPG,), jnp.float32),
            pltpu.VMEM((CH, D), jnp.float32),
            pltpu.VMEM((CH, D), jnp.float32),
            pltpu.VMEM((CH, D), jnp.float32),
            pltpu.VMEM((CH, D), jnp.float32),
            pltpu.VMEM_SHARED((NPAD, D), jnp.float32),
            pltpu.SemaphoreType.DMA,
            pltpu.SemaphoreType.DMA,
            pltpu.SemaphoreType.DMA,
            pltpu.SemaphoreType.DMA,
            pltpu.SemaphoreType.DMA,
            pltpu.SemaphoreType.DMA,
            pltpu.SemaphoreType.DMA,
        ],
    )(_sc_scatter_kernel)
    src_adj = (edge_index[:, 0, :] + (jnp.arange(T, dtype=jnp.int32) * N)[:, None]
               ).reshape(-1)
    dst2 = edge_index[:, 1, :].reshape(T * NTILES * NG, CPG, CH)
    ew_flat = edge_weight.reshape(-1)
    zrows = jnp.zeros((CH, D), jnp.float32)
    ne_flat = node_embs.reshape(T * N, D)
    partials = sc_scatter(src_adj, dst2, ew_flat, ne_flat, zrows)

    full = pl.BlockSpec((D, D), lambda t, nb: (0, 0))
    out = pl.pallas_call(
        _tc_merge_kernel,
        grid=(T, NB),
        in_specs=[pl.BlockSpec((2, 1, BN, D), lambda t, nb: (0, t, nb, 0))]
                 + [full] * 13,
        out_specs=pl.BlockSpec((1, BN, D), lambda t, nb: (t, nb, 0)),
        out_shape=jax.ShapeDtypeStruct((T, N, D), jnp.float32),
        scratch_shapes=[pltpu.VMEM((D, D), jnp.float32),
                        pltpu.VMEM((D, D), jnp.float32)],
    )(partials, gcn_init_weights, Wi, Ui, bi, Wf, Uf, bf, Wo, Uo, bo, Wg, Ug, bg)
    return out
